# baseline (speedup 1.0000x reference)
import functools

import jax
import jax.numpy as jnp
from jax import lax
from jax.experimental import pallas as pl
from jax.experimental.pallas import tpu as pltpu
from jax.experimental.pallas import tpu_sc as plsc


def _sc_gather(table, idx, B, D, NC, NS):
    NW = NC * NS
    b_per_w = B // NW
    mesh = plsc.VectorSubcoreMesh(core_axis_name="c", subcore_axis_name="s")

    @functools.partial(
        pl.kernel,
        mesh=mesh,
        out_type=jax.ShapeDtypeStruct((B, D), jnp.float32),
        scratch_types=[
            pltpu.VMEM((b_per_w,), jnp.int32),
            pltpu.VMEM((b_per_w, D), jnp.float32),
            pltpu.SemaphoreType.DMA,
        ],
    )
    def k(table_hbm, idx_hbm, out_hbm, idx_v, rows_v, sem):
        wid = lax.axis_index("s") * NC + lax.axis_index("c")
        base = wid * b_per_w
        pltpu.sync_copy(idx_hbm.at[pl.ds(base, b_per_w)], idx_v)
        pltpu.async_copy(table_hbm.at[idx_v], rows_v, sem).wait()
        pltpu.sync_copy(rows_v, out_hbm.at[pl.ds(base, b_per_w)])

    return k(table, idx)


def _tc_assemble(gathered, prefix2d, suffix2d, B, PRE, CLS, SUF, BB):
    W = PRE + CLS + SUF

    def body(g_ref, p_ref, s_ref, o_ref):
        o_ref[:, 0:PRE] = jnp.broadcast_to(p_ref[...], (BB, PRE))
        o_ref[:, PRE:PRE + CLS] = g_ref[...]
        o_ref[:, PRE + CLS:W] = jnp.broadcast_to(s_ref[...], (BB, SUF))

    return pl.pallas_call(
        body,
        grid=(B // BB,),
        in_specs=[
            pl.BlockSpec((BB, CLS), lambda i: (i, 0)),
            pl.BlockSpec((1, PRE), lambda i: (0, 0)),
            pl.BlockSpec((1, SUF), lambda i: (0, 0)),
        ],
        out_specs=pl.BlockSpec((BB, W), lambda i: (i, 0)),
        out_shape=jax.ShapeDtypeStruct((B, W), jnp.float32),
        compiler_params=pltpu.CompilerParams(
            dimension_semantics=("parallel",),
        ),
    )(gathered, prefix2d, suffix2d)


def kernel(label, cls_ctx, token_prefix, token_suffix):
    B = label.shape[0]
    V, NT, D = cls_ctx.shape
    PL_, SL = token_prefix.shape[1], token_suffix.shape[1]
    CLS = NT * D
    PRE = PL_ * D
    SUF = SL * D

    table = cls_ctx.reshape(V, CLS)
    idx = label.astype(jnp.int32)

    info = plsc.get_sparse_core_info()
    gathered = _sc_gather(table, idx, B, CLS, info.num_cores, info.num_subcores)

    out2d = _tc_assemble(
        gathered,
        token_prefix.reshape(1, PRE),
        token_suffix.reshape(1, SUF),
        B, PRE, CLS, SUF, BB=8,
    )
    return out2d.reshape(B, PL_ + NT + SL, D)


# speedup vs baseline: 4.1977x; 4.1977x over previous
import functools

import jax
import jax.numpy as jnp
from jax import lax
from jax.experimental import pallas as pl
from jax.experimental.pallas import tpu as pltpu
from jax.experimental.pallas import tpu_sc as plsc


def _sc_gather(table, idx, B, NT, D, NC, NS):
    NW = NC * NS
    b_per_w = B // NW
    mesh = plsc.VectorSubcoreMesh(core_axis_name="c", subcore_axis_name="s")

    @functools.partial(
        pl.kernel,
        mesh=mesh,
        out_type=jax.ShapeDtypeStruct((B, NT, D), jnp.float32),
        scratch_types=[
            pltpu.VMEM((b_per_w,), jnp.int32),
            pltpu.VMEM((b_per_w, NT, D), jnp.float32),
            pltpu.SemaphoreType.DMA,
        ],
    )
    def k(table_hbm, idx_hbm, out_hbm, idx_v, rows_v, sem):
        wid = lax.axis_index("s") * NC + lax.axis_index("c")
        base = wid * b_per_w
        pltpu.sync_copy(idx_hbm.at[pl.ds(base, b_per_w)], idx_v)
        pltpu.async_copy(table_hbm.at[idx_v], rows_v, sem).wait()
        pltpu.sync_copy(rows_v, out_hbm.at[pl.ds(base, b_per_w)])

    return k(table, idx)


def _tc_assemble(gathered, prefix, suffix, B, NT, D, PL_, SL, BB):
    T = PL_ + NT + SL

    def body(g_ref, p_ref, s_ref, o_ref):
        o_ref[:, 0:PL_, :] = jnp.broadcast_to(p_ref[...], (BB, PL_, D))
        o_ref[:, PL_:PL_ + NT, :] = g_ref[...]
        o_ref[:, PL_ + NT:T, :] = jnp.broadcast_to(s_ref[...], (BB, SL, D))

    return pl.pallas_call(
        body,
        grid=(B // BB,),
        in_specs=[
            pl.BlockSpec((BB, NT, D), lambda i: (i, 0, 0)),
            pl.BlockSpec((1, PL_, D), lambda i: (0, 0, 0)),
            pl.BlockSpec((1, SL, D), lambda i: (0, 0, 0)),
        ],
        out_specs=pl.BlockSpec((BB, T, D), lambda i: (i, 0, 0)),
        out_shape=jax.ShapeDtypeStruct((B, T, D), jnp.float32),
        compiler_params=pltpu.CompilerParams(
            dimension_semantics=("parallel",),
        ),
    )(gathered, prefix, suffix)


def kernel(label, cls_ctx, token_prefix, token_suffix):
    B = label.shape[0]
    V, NT, D = cls_ctx.shape
    PL_, SL = token_prefix.shape[1], token_suffix.shape[1]

    idx = label.astype(jnp.int32)
    info = plsc.get_sparse_core_info()
    gathered = _sc_gather(cls_ctx, idx, B, NT, D,
                          info.num_cores, info.num_subcores)
    return _tc_assemble(gathered, token_prefix, token_suffix,
                        B, NT, D, PL_, SL, BB=8)


# speedup vs baseline: 5.1762x; 1.2331x over previous
import functools

import jax
import jax.numpy as jnp
from jax import lax
from jax.experimental import pallas as pl
from jax.experimental.pallas import tpu as pltpu
from jax.experimental.pallas import tpu_sc as plsc


def _sc_gather(table, idx, B, NT, D, NC, NS):
    NW = NC * NS
    b_per_w = B // NW
    mesh = plsc.VectorSubcoreMesh(core_axis_name="c", subcore_axis_name="s")

    @functools.partial(
        pl.kernel,
        mesh=mesh,
        out_type=jax.ShapeDtypeStruct((B, NT, D), jnp.float32),
        scratch_types=[
            pltpu.VMEM((b_per_w,), jnp.int32),
            pltpu.VMEM((b_per_w, NT, D), jnp.float32),
            pltpu.SemaphoreType.DMA,
        ],
    )
    def k(table_hbm, idx_hbm, out_hbm, idx_v, rows_v, sem):
        wid = lax.axis_index("s") * NC + lax.axis_index("c")
        base = wid * b_per_w
        pltpu.sync_copy(idx_hbm.at[pl.ds(base, b_per_w)], idx_v)
        pltpu.async_copy(table_hbm.at[idx_v], rows_v, sem).wait()
        pltpu.sync_copy(rows_v, out_hbm.at[pl.ds(base, b_per_w)])

    return k(table, idx)


def _tc_assemble(gathered, prefix, suffix, B, NT, D, PL_, SL, BB):
    T = PL_ + NT + SL

    def body(g_ref, p_ref, s_ref, o_ref):
        o_ref[:, 0:PL_, :] = jnp.broadcast_to(p_ref[...], (BB, PL_, D))
        o_ref[:, PL_:PL_ + NT, :] = g_ref[...]
        o_ref[:, PL_ + NT:T, :] = jnp.broadcast_to(s_ref[...], (BB, SL, D))

    return pl.pallas_call(
        body,
        grid=(B // BB,),
        in_specs=[
            pl.BlockSpec((BB, NT, D), lambda i: (i, 0, 0)),
            pl.BlockSpec((1, PL_, D), lambda i: (0, 0, 0)),
            pl.BlockSpec((1, SL, D), lambda i: (0, 0, 0)),
        ],
        out_specs=pl.BlockSpec((BB, T, D), lambda i: (i, 0, 0)),
        out_shape=jax.ShapeDtypeStruct((B, T, D), jnp.float32),
        compiler_params=pltpu.CompilerParams(
            dimension_semantics=("parallel",),
        ),
    )(gathered, prefix, suffix)


def kernel(label, cls_ctx, token_prefix, token_suffix):
    B = label.shape[0]
    V, NT, D = cls_ctx.shape
    PL_, SL = token_prefix.shape[1], token_suffix.shape[1]

    idx = label.astype(jnp.int32)
    info = plsc.get_sparse_core_info()
    gathered = _sc_gather(cls_ctx, idx, B, NT, D,
                          info.num_cores, info.num_subcores)
    return _tc_assemble(gathered, token_prefix, token_suffix,
                        B, NT, D, PL_, SL, BB=32)


# speedup vs baseline: 5.2199x; 1.0085x over previous
import functools

import jax
import jax.numpy as jnp
from jax import lax
from jax.experimental import pallas as pl
from jax.experimental.pallas import tpu as pltpu
from jax.experimental.pallas import tpu_sc as plsc


def _sc_gather(table, idx, B, NT, D, NC, NS):
    NW = NC * NS
    b_per_w = B // NW
    mesh = plsc.VectorSubcoreMesh(core_axis_name="c", subcore_axis_name="s")

    @functools.partial(
        pl.kernel,
        mesh=mesh,
        out_type=jax.ShapeDtypeStruct((B, NT, D), jnp.float32),
        scratch_types=[
            pltpu.VMEM((b_per_w,), jnp.int32),
            pltpu.VMEM((b_per_w, NT, D), jnp.float32),
            pltpu.SemaphoreType.DMA,
        ],
    )
    def k(table_hbm, idx_hbm, out_hbm, idx_v, rows_v, sem):
        wid = lax.axis_index("s") * NC + lax.axis_index("c")
        base = wid * b_per_w
        pltpu.sync_copy(idx_hbm.at[pl.ds(base, b_per_w)], idx_v)
        pltpu.async_copy(table_hbm.at[idx_v], rows_v, sem).wait()
        pltpu.sync_copy(rows_v, out_hbm.at[pl.ds(base, b_per_w)])

    return k(table, idx)


def _tc_assemble(gathered, prefix, suffix, B, NT, D, PL_, SL, BB):
    T = PL_ + NT + SL

    def body(g_ref, p_ref, s_ref, o_ref):
        o_ref[:, 0:PL_, :] = jnp.broadcast_to(p_ref[...], (BB, PL_, D))
        o_ref[:, PL_:PL_ + NT, :] = g_ref[...]
        o_ref[:, PL_ + NT:T, :] = jnp.broadcast_to(s_ref[...], (BB, SL, D))

    return pl.pallas_call(
        body,
        grid=(B // BB,),
        in_specs=[
            pl.BlockSpec((BB, NT, D), lambda i: (i, 0, 0)),
            pl.BlockSpec((1, PL_, D), lambda i: (0, 0, 0)),
            pl.BlockSpec((1, SL, D), lambda i: (0, 0, 0)),
        ],
        out_specs=pl.BlockSpec((BB, T, D), lambda i: (i, 0, 0)),
        out_shape=jax.ShapeDtypeStruct((B, T, D), jnp.float32),
        compiler_params=pltpu.CompilerParams(
            dimension_semantics=("parallel",),
        ),
    )(gathered, prefix, suffix)


def kernel(label, cls_ctx, token_prefix, token_suffix):
    B = label.shape[0]
    V, NT, D = cls_ctx.shape
    PL_, SL = token_prefix.shape[1], token_suffix.shape[1]

    idx = label.astype(jnp.int32)
    info = plsc.get_sparse_core_info()
    gathered = _sc_gather(cls_ctx, idx, B, NT, D,
                          info.num_cores, info.num_subcores)
    return _tc_assemble(gathered, token_prefix, token_suffix,
                        B, NT, D, PL_, SL, BB=64)


# speedup vs baseline: 5.2345x; 1.0028x over previous
import functools

import jax
import jax.numpy as jnp
from jax import lax
from jax.experimental import pallas as pl
from jax.experimental.pallas import tpu as pltpu
from jax.experimental.pallas import tpu_sc as plsc


def _sc_gather(table, idx, B, NT, D, NC, NS):
    NW = NC * NS
    b_per_w = B // NW
    mesh = plsc.VectorSubcoreMesh(core_axis_name="c", subcore_axis_name="s")

    @functools.partial(
        pl.kernel,
        mesh=mesh,
        out_type=jax.ShapeDtypeStruct((B, NT, D), jnp.float32),
        scratch_types=[
            pltpu.VMEM((b_per_w,), jnp.int32),
            pltpu.VMEM((b_per_w, NT, D), jnp.float32),
            pltpu.SemaphoreType.DMA,
        ],
    )
    def k(table_hbm, idx_hbm, out_hbm, idx_v, rows_v, sem):
        wid = lax.axis_index("s") * NC + lax.axis_index("c")
        base = wid * b_per_w
        pltpu.sync_copy(idx_hbm.at[pl.ds(base, b_per_w)], idx_v)
        pltpu.async_copy(table_hbm.at[idx_v], rows_v, sem).wait()
        pltpu.sync_copy(rows_v, out_hbm.at[pl.ds(base, b_per_w)])

    return k(table, idx)


def _tc_assemble(gathered, prefix, suffix, B, NT, D, PL_, SL, BB):
    T = PL_ + NT + SL

    def body(g_ref, p_ref, s_ref, o_ref):
        o_ref[:, 0:PL_, :] = jnp.broadcast_to(p_ref[...], (BB, PL_, D))
        o_ref[:, PL_:PL_ + NT, :] = g_ref[...]
        o_ref[:, PL_ + NT:T, :] = jnp.broadcast_to(s_ref[...], (BB, SL, D))

    return pl.pallas_call(
        body,
        grid=(B // BB,),
        in_specs=[
            pl.BlockSpec((BB, NT, D), lambda i: (i, 0, 0)),
            pl.BlockSpec((1, PL_, D), lambda i: (0, 0, 0)),
            pl.BlockSpec((1, SL, D), lambda i: (0, 0, 0)),
        ],
        out_specs=pl.BlockSpec((BB, T, D), lambda i: (i, 0, 0)),
        out_shape=jax.ShapeDtypeStruct((B, T, D), jnp.float32),
        compiler_params=pltpu.CompilerParams(
            dimension_semantics=("parallel",),
        ),
    )(gathered, prefix, suffix)


def kernel(label, cls_ctx, token_prefix, token_suffix):
    B = label.shape[0]
    V, NT, D = cls_ctx.shape
    PL_, SL = token_prefix.shape[1], token_suffix.shape[1]

    idx = label.astype(jnp.int32)
    info = plsc.get_sparse_core_info()
    gathered = _sc_gather(cls_ctx, idx, B, NT, D,
                          info.num_cores, info.num_subcores)
    return _tc_assemble(gathered, token_prefix, token_suffix,
                        B, NT, D, PL_, SL, BB=128)
